# Initial kernel scaffold; baseline (speedup 1.0000x reference)
import jax
import jax.numpy as jnp
from jax.experimental import pallas as pl


def kernel(x, edge_index, edge_weight, weight, bias):
    raise NotImplementedError("write your pallas kernel here")


# speedup vs baseline: 4.5330x; 4.5330x over previous
import jax
import jax.numpy as jnp
from jax.experimental import pallas as pl


def _matmul_body(l_ref, x_ref, o_ref):
    o_ref[...] = jnp.dot(
        l_ref[...], x_ref[...], preferred_element_type=jnp.float32
    ).astype(o_ref.dtype)


def _spmm_dense(lb, xb, tm):
    vp = lb.shape[0]
    n = xb.shape[1]
    return pl.pallas_call(
        _matmul_body,
        grid=(vp // tm,),
        in_specs=[
            pl.BlockSpec((tm, vp), lambda i: (i, 0)),
            pl.BlockSpec((vp, n), lambda i: (0, 0)),
        ],
        out_specs=pl.BlockSpec((tm, n), lambda i: (i, 0)),
        out_shape=jax.ShapeDtypeStruct((vp, n), jnp.bfloat16),
    )(lb, xb)


def _combine_body(x0_ref, x1_ref, x2_ref, w0_ref, wb_ref, b_ref, o_ref):
    acc = jnp.dot(x0_ref[...], w0_ref[...], preferred_element_type=jnp.float32)
    acc = acc + jnp.dot(x1_ref[...], wb_ref[0], preferred_element_type=jnp.float32)
    acc = acc + jnp.dot(x2_ref[...], wb_ref[1], preferred_element_type=jnp.float32)
    o_ref[...] = acc + b_ref[...]


def _combine(x0, x1b, x2b, w0, wb, bias2d, tv):
    vp, bc = x0.shape
    c = w0.shape[0]
    f = w0.shape[1]
    nb = bc // c
    return pl.pallas_call(
        _combine_body,
        grid=(vp // tv, nb),
        in_specs=[
            pl.BlockSpec((tv, c), lambda i, b: (i, b)),
            pl.BlockSpec((tv, c), lambda i, b: (i, b)),
            pl.BlockSpec((tv, c), lambda i, b: (i, b)),
            pl.BlockSpec((c, f), lambda i, b: (0, 0)),
            pl.BlockSpec((2, c, f), lambda i, b: (0, 0, 0)),
            pl.BlockSpec((1, f), lambda i, b: (0, 0)),
        ],
        out_specs=pl.BlockSpec((tv, f), lambda i, b: (i, b)),
        out_shape=jax.ShapeDtypeStruct((vp, nb * f), jnp.float32),
    )(x0, x1b, x2b, w0, wb, bias2d)


def kernel(x, edge_index, edge_weight, weight, bias):
    b, c, vn = x.shape
    k = weight.shape[0] // c
    f = weight.shape[1]
    assert k == 3
    vp = -(-vn // 1024) * 1024

    src = edge_index[0].astype(jnp.int32)
    dst = edge_index[1].astype(jnp.int32)

    x0 = jnp.transpose(x, (2, 0, 1)).reshape(vn, b * c)
    x0 = jnp.pad(x0, ((0, vp - vn), (0, 0)))
    x0b = x0.astype(jnp.bfloat16)

    flat = dst * vp + src
    lf = jnp.zeros((vp * vp,), jnp.float32).at[flat].add(edge_weight)
    lb = lf.reshape(vp, vp).astype(jnp.bfloat16)

    tm = 512 if vp % 512 == 0 else vp
    x1b = _spmm_dense(lb, x0b, tm)
    x2b = _spmm_dense(lb, x1b, tm)

    wk = weight.reshape(c, k, f)
    w0 = wk[:, 0, :] - wk[:, 2, :]
    wb = jnp.stack([wk[:, 1, :], 2.0 * wk[:, 2, :]]).astype(jnp.bfloat16)

    tv = 1024 if vp % 1024 == 0 else vp
    outp = _combine(x0, x1b, x2b, w0, wb, bias.reshape(1, f), tv)
    return outp[:vn].reshape(vn, b, f).transpose(1, 2, 0)


# speedup vs baseline: 4.9225x; 1.0859x over previous
import functools

import jax
import jax.numpy as jnp
from jax.experimental import pallas as pl


def _prep_body(vn, x_ref, w1t_ref, w2t_ref, w0t_ref, b_ref, y1_ref, y2_ref, base_ref):
    nb = x_ref.shape[0] // w1t_ref.shape[0]
    c = w1t_ref.shape[0]
    xf = x_ref[...]
    j = pl.program_id(0)
    col = jax.lax.broadcasted_iota(jnp.int32, (1, x_ref.shape[1]), 1)
    valid = (col + j * x_ref.shape[1]) < vn
    xf = jnp.where(valid, xf, 0.0)
    xb = xf.astype(jnp.bfloat16)
    for b in range(nb):
        sl = slice(b * c, (b + 1) * c)
        y1_ref[sl, :] = jnp.dot(
            w1t_ref[...], xb[sl, :], preferred_element_type=jnp.float32
        ).astype(jnp.bfloat16)
        y2_ref[sl, :] = jnp.dot(
            w2t_ref[...], xb[sl, :], preferred_element_type=jnp.float32
        ).astype(jnp.bfloat16)
        base_ref[sl, :] = (
            jnp.dot(w0t_ref[...], xf[sl, :], preferred_element_type=jnp.float32)
            + b_ref[0, sl].reshape(c, 1)
        )


def _prep(x0t, w1t, w2t, w0t, bias_bc, vp, tv):
    m, vn = x0t.shape
    c = w1t.shape[0]
    grid = (vp // tv,)
    return pl.pallas_call(
        functools.partial(_prep_body, vn),
        grid=grid,
        in_specs=[
            pl.BlockSpec((m, tv), lambda j: (0, j)),
            pl.BlockSpec((c, c), lambda j: (0, 0)),
            pl.BlockSpec((c, c), lambda j: (0, 0)),
            pl.BlockSpec((c, c), lambda j: (0, 0)),
            pl.BlockSpec((1, m), lambda j: (0, 0)),
        ],
        out_specs=[
            pl.BlockSpec((m, tv), lambda j: (0, j)),
            pl.BlockSpec((m, tv), lambda j: (0, j)),
            pl.BlockSpec((m, tv), lambda j: (0, j)),
        ],
        out_shape=[
            jax.ShapeDtypeStruct((m, vp), jnp.bfloat16),
            jax.ShapeDtypeStruct((m, vp), jnp.bfloat16),
            jax.ShapeDtypeStruct((m, vp), jnp.float32),
        ],
    )(x0t, w1t, w2t, w0t, bias_bc)


def _hop1_body(lhs_ref, lt_ref, add_ref, o_ref):
    acc = jnp.dot(
        lhs_ref[...], lt_ref[...].astype(jnp.bfloat16),
        preferred_element_type=jnp.float32,
    )
    o_ref[...] = (acc + add_ref[...]).astype(jnp.bfloat16)


def _hop2_body(lhs_ref, lt_ref, add_ref, o_ref):
    acc = jnp.dot(
        lhs_ref[...], lt_ref[...].astype(jnp.bfloat16),
        preferred_element_type=jnp.float32,
    )
    o_ref[...] = acc + add_ref[...]


def _hop(lhs, ltf, add, out_dtype, body, tn):
    m, vp = lhs.shape
    return pl.pallas_call(
        body,
        grid=(vp // tn,),
        in_specs=[
            pl.BlockSpec((m, vp), lambda j: (0, 0)),
            pl.BlockSpec((vp, tn), lambda j: (0, j)),
            pl.BlockSpec((m, tn), lambda j: (0, j)),
        ],
        out_specs=pl.BlockSpec((m, tn), lambda j: (0, j)),
        out_shape=jax.ShapeDtypeStruct((m, vp), out_dtype),
    )(lhs, ltf, add)


def kernel(x, edge_index, edge_weight, weight, bias):
    b, c, vn = x.shape
    k = weight.shape[0] // c
    f = weight.shape[1]
    assert k == 3 and f == c
    vp = -(-vn // 1024) * 1024

    src = edge_index[0].astype(jnp.int32)
    dst = edge_index[1].astype(jnp.int32)

    x0t = x.reshape(b * c, vn)

    flat = src * vp + dst
    ltf = jnp.zeros((vp * vp,), jnp.float32).at[flat].add(edge_weight)
    ltf = ltf.reshape(vp, vp)

    wk = weight.reshape(c, k, f)
    w1t = wk[:, 1, :].T.astype(jnp.bfloat16)
    w2t = (2.0 * wk[:, 2, :]).T.astype(jnp.bfloat16)
    w0t = (wk[:, 0, :] - wk[:, 2, :]).T
    bias_bc = jnp.tile(bias.reshape(1, f), (1, b))

    y1t, y2t, baset = _prep(x0t, w1t, w2t, w0t, bias_bc, vp, tv=1024)

    ut = _hop(y2t, ltf, y1t, jnp.bfloat16, _hop1_body, tn=256)
    outt = _hop(ut, ltf, baset, jnp.float32, _hop2_body, tn=256)

    return outt[:, :vn].reshape(b, f, vn)
